# Initial kernel scaffold; baseline (speedup 1.0000x reference)
import jax
import jax.numpy as jnp
from jax.experimental import pallas as pl


def kernel(inputs, targets, V):
    raise NotImplementedError("write your pallas kernel here")


# speedup vs baseline: 15.8879x; 15.8879x over previous
import functools
import math

import jax
import jax.numpy as jnp
from jax.experimental import pallas as pl
from jax.experimental.pallas import tpu as pltpu

_NUM_CLASSES = 100000
_F = 64
_B = 1024
_M = 0.9
_LN_M = math.log(_M)
_T = 1000
_GRID = _NUM_CLASSES // _T


def _body(x_ref, tcol_ref, v_ref, vnew_ref, loss_ref,
          macc, sacc, tacc, vrows):
    i = pl.program_id(0)
    x = x_ref[...]
    t_col = tcol_ref[...]

    @pl.when(i == 0)
    def _prep():
        macc[...] = jnp.full((_B, 1), -jnp.inf, jnp.float32)
        sacc[...] = jnp.zeros((_B, 1), jnp.float32)
        tacc[...] = jnp.zeros((_B, 1), jnp.float32)
        ii = jax.lax.broadcasted_iota(jnp.int32, (_B, _B), 0)
        jj = jax.lax.broadcasted_iota(jnp.int32, (_B, _B), 1)
        ident = jnp.where(ii == jj, 1.0, 0.0)
        t_f = t_col.astype(jnp.float32)
        t_row = jax.lax.dot_general(
            t_f, ident, (((0,), (0,)), ((), ())),
            preferred_element_type=jnp.float32)
        eq = t_f == t_row
        after = jnp.sum(jnp.where(eq & (jj > ii), 1.0, 0.0), axis=1, keepdims=True)
        w = (1.0 - _M) * jnp.exp(after * _LN_M)
        vrows[...] = (w * x).astype(jnp.bfloat16)

    v = v_ref[...]
    logits = jax.lax.dot_general(
        x.astype(jnp.bfloat16), v.astype(jnp.bfloat16),
        (((1,), (1,)), ((), ())), preferred_element_type=jnp.float32)

    m_old = macc[...]
    m_new = jnp.maximum(m_old, jnp.max(logits, axis=1, keepdims=True))
    macc[...] = m_new
    sacc[...] = (sacc[...] * jnp.exp(m_old - m_new)
                 + jnp.sum(jnp.exp(logits - m_new), axis=1, keepdims=True))

    col_ids = jax.lax.broadcasted_iota(jnp.int32, (_B, _T), 1) + i * _T
    onehot = col_ids == t_col
    tacc[...] += jnp.sum(jnp.where(onehot, logits, 0.0), axis=1, keepdims=True)

    oh = jnp.where(onehot, 1.0, 0.0).astype(jnp.bfloat16)
    contrib = jax.lax.dot_general(
        oh, vrows[...], (((0,), (0,)), ((), ())),
        preferred_element_type=jnp.float32)
    counts = jax.lax.dot_general(
        oh, jnp.ones((_B, 1), jnp.bfloat16), (((0,), (0,)), ((), ())),
        preferred_element_type=jnp.float32)
    vnew_ref[...] = v * jnp.exp(counts * _LN_M) + contrib

    @pl.when(i == _GRID - 1)
    def _fin():
        loss_ref[...] = jnp.mean(
            macc[...] + jnp.log(sacc[...]) - tacc[...]).reshape(1, 1)


@jax.jit
def kernel(inputs, targets, V):
    t_col = targets.astype(jnp.int32).reshape(_B, 1)
    vnew, loss = pl.pallas_call(
        _body,
        grid=(_GRID,),
        in_specs=[
            pl.BlockSpec((_B, _F), lambda i: (0, 0)),
            pl.BlockSpec((_B, 1), lambda i: (0, 0)),
            pl.BlockSpec((_T, _F), lambda i: (i, 0)),
        ],
        out_specs=[
            pl.BlockSpec((_T, _F), lambda i: (i, 0)),
            pl.BlockSpec((1, 1), lambda i: (0, 0)),
        ],
        out_shape=[
            jax.ShapeDtypeStruct((_NUM_CLASSES, _F), jnp.float32),
            jax.ShapeDtypeStruct((1, 1), jnp.float32),
        ],
        scratch_shapes=[
            pltpu.VMEM((_B, 1), jnp.float32),
            pltpu.VMEM((_B, 1), jnp.float32),
            pltpu.VMEM((_B, 1), jnp.float32),
            pltpu.VMEM((_B, _F), jnp.bfloat16),
        ],
    )(inputs, t_col, V)
    return (loss.reshape(()), vnew)
